# Initial kernel scaffold; baseline (speedup 1.0000x reference)
import jax
import jax.numpy as jnp
from jax.experimental import pallas as pl


def kernel(X, W):
    raise NotImplementedError("write your pallas kernel here")


# speedup vs baseline: 10.7202x; 10.7202x over previous
import functools

import jax
import jax.numpy as jnp
from jax import lax
from jax.experimental import pallas as pl
from jax.experimental.pallas import tpu as pltpu
from jax.experimental.pallas import tpu_sc as plsc

VOC_SIZE = 1000000
EMB_DIM = 32
BATCH = 16384
HIST_LEN = 200
HALF = 100

_info = plsc.get_sparse_core_info()
NC = _info.num_cores
NS = _info.num_subcores
NW = NC * NS
ROWS_PER_W = BATCH // NW
HALVES_PER_W = 2 * ROWS_PER_W
CHUNK_H = 4
NCHUNKS = HALVES_PER_W // CHUNK_H


def _count_zeros(idx_ref, r):
    lane = lax.iota(jnp.int32, 16)
    one = jnp.ones((16,), jnp.int32)
    nil = jnp.zeros((16,), jnp.int32)
    cnt = nil
    for o in (0, 16, 32, 48, 64, 80):
        v = idx_ref[r, pl.ds(o, 16)]
        cnt = cnt + jnp.where(v == 0, one, nil)
    v = idx_ref[r, pl.ds(84, 16)]
    cnt = cnt + jnp.where(jnp.logical_and(v == 0, lane >= 12), one, nil)
    return cnt


def _hsum16(vec, scratch_ref):
    lane = lax.iota(jnp.int32, 16)
    for sh in (8, 4, 2, 1):
        scratch_ref[...] = vec
        vec = vec + plsc.load_gather(scratch_ref, [lane ^ sh])
    return vec


def _body(x2_hbm, w_hbm, out_hbm, idx_v, rows_bufs, out_v, w0_v, hs_v, sem):
    wid = lax.axis_index("s") * NC + lax.axis_index("c")
    base_h = wid * HALVES_PER_W

    pltpu.sync_copy(w_hbm.at[pl.ds(0, 8)], w0_v)
    w0a = w0_v[0, pl.ds(0, 16)]
    w0b = w0_v[0, pl.ds(16, 16)]

    zero = jnp.zeros((16,), jnp.float32)

    def chunk(ci, carry):
        h0 = base_h + ci * CHUNK_H
        pltpu.sync_copy(x2_hbm.at[pl.ds(h0, CHUNK_H)], idx_v)
        copies = []
        for r in range(CHUNK_H):
            copies.append(
                pltpu.async_copy(w_hbm.at[idx_v.at[r]], rows_bufs[r], sem)
            )
        for c in copies:
            c.wait()
        for pair in range(CHUNK_H // 2):
            acc0 = zero
            acc1 = zero
            nz = None
            for r in (2 * pair, 2 * pair + 1):
                rv = rows_bufs[r]
                for i in range(HALF):
                    acc0 = acc0 + rv[i, pl.ds(0, 16)]
                    acc1 = acc1 + rv[i, pl.ds(16, 16)]
                zc = _count_zeros(idx_v, r)
                nz = zc if nz is None else nz + zc
            nz = _hsum16(nz, hs_v)
            nzf = nz.astype(jnp.float32)
            cntf = (HIST_LEN - nz).astype(jnp.float32)
            orow = ci * (CHUNK_H // 2) + pair
            out_v[orow, pl.ds(0, 16)] = (acc0 - nzf * w0a) / cntf
            out_v[orow, pl.ds(16, 16)] = (acc1 - nzf * w0b) / cntf
        return carry

    lax.fori_loop(0, NCHUNKS, chunk, 0)
    pltpu.sync_copy(out_v, out_hbm.at[pl.ds(wid * ROWS_PER_W, ROWS_PER_W)])


@functools.partial(jax.jit, donate_argnums=())
def kernel(X, W):
    X2 = X.astype(jnp.int32).reshape(BATCH * 2, HALF)
    mesh = plsc.VectorSubcoreMesh(core_axis_name="c", subcore_axis_name="s")
    k = pl.kernel(
        _body,
        mesh=mesh,
        out_type=jax.ShapeDtypeStruct((BATCH, EMB_DIM), jnp.float32),
        scratch_types=[
            pltpu.VMEM((CHUNK_H, HALF), jnp.int32),
            [pltpu.VMEM((HALF, EMB_DIM), jnp.float32) for _ in range(CHUNK_H)],
            pltpu.VMEM((ROWS_PER_W, EMB_DIM), jnp.float32),
            pltpu.VMEM((8, EMB_DIM), jnp.float32),
            pltpu.VMEM((16,), jnp.int32),
            pltpu.SemaphoreType.DMA,
        ],
        compiler_params=pltpu.CompilerParams(
            needs_layout_passes=False, use_tc_tiling_on_sc=False
        ),
    )
    return k(X2, W)


# speedup vs baseline: 12.9010x; 1.2034x over previous
import functools

import jax
import jax.numpy as jnp
from jax import lax
from jax.experimental import pallas as pl
from jax.experimental.pallas import tpu as pltpu
from jax.experimental.pallas import tpu_sc as plsc

VOC_SIZE = 1000000
EMB_DIM = 32
BATCH = 16384
HIST_LEN = 200
HALF = 100

_info = plsc.get_sparse_core_info()
NC = _info.num_cores
NS = _info.num_subcores
NW = NC * NS
ROWS_PER_W = BATCH // NW
HALVES_PER_W = 2 * ROWS_PER_W
IDX_CHUNK = 512
NPHASE = HALVES_PER_W // IDX_CHUNK
NBUF = 8
NGROUP = IDX_CHUNK // NBUF


def _count_zeros(idx_ref, r):
    lane = lax.iota(jnp.int32, 16)
    one = jnp.ones((16,), jnp.int32)
    nil = jnp.zeros((16,), jnp.int32)
    cnt = nil
    for o in (0, 16, 32, 48, 64, 80):
        v = idx_ref[r, pl.ds(o, 16)]
        cnt = cnt + jnp.where(v == 0, one, nil)
    v = idx_ref[r, pl.ds(84, 16)]
    cnt = cnt + jnp.where(jnp.logical_and(v == 0, lane >= 12), one, nil)
    return cnt


def _hsum16(vec, scratch_ref):
    lane = lax.iota(jnp.int32, 16)
    for sh in (8, 4, 2, 1):
        scratch_ref[...] = vec
        vec = vec + plsc.load_gather(scratch_ref, [lane ^ sh])
    return vec


def _body(x2_hbm, w_hbm, out_hbm, idx_v, bufs, out_v, w0_v, hs_v, sems):
    wid = lax.axis_index("s") * NC + lax.axis_index("c")
    base_h = wid * HALVES_PER_W

    pltpu.sync_copy(w_hbm.at[pl.ds(0, 8)], w0_v)
    w0a = w0_v[0, pl.ds(0, 16)]
    w0b = w0_v[0, pl.ds(16, 16)]

    zero = jnp.zeros((16,), jnp.float32)

    def fire(h, b):
        pltpu.async_copy(w_hbm.at[idx_v.at[h]], bufs[b], sems[b])

    def drain(h, b):
        pltpu.make_async_copy(w_hbm.at[idx_v.at[h]], bufs[b], sems[b]).wait()

    for p in range(NPHASE):
        pltpu.sync_copy(
            x2_hbm.at[pl.ds(base_h + p * IDX_CHUNK, IDX_CHUNK)], idx_v
        )
        for b in range(NBUF):
            fire(b, b)

        def group(g, carry, p=p):
            h0 = g * NBUF
            more = g < NGROUP - 1
            for pairb in range(NBUF // 2):
                acc0 = zero
                acc1 = zero
                nz = None
                for b in (2 * pairb, 2 * pairb + 1):
                    h = h0 + b
                    drain(h, b)
                    rv = bufs[b]
                    for i in range(HALF):
                        acc0 = acc0 + rv[i, pl.ds(0, 16)]
                        acc1 = acc1 + rv[i, pl.ds(16, 16)]
                    zc = _count_zeros(idx_v, h)
                    nz = zc if nz is None else nz + zc

                    @pl.when(more)
                    def _(h=h, b=b):
                        fire(h + NBUF, b)

                nz = _hsum16(nz, hs_v)
                nzf = nz.astype(jnp.float32)
                cntf = (HIST_LEN - nz).astype(jnp.float32)
                orow = p * (IDX_CHUNK // 2) + (h0 // 2) + pairb
                out_v[orow, pl.ds(0, 16)] = (acc0 - nzf * w0a) / cntf
                out_v[orow, pl.ds(16, 16)] = (acc1 - nzf * w0b) / cntf
            return carry

        lax.fori_loop(0, NGROUP, group, 0)

    pltpu.sync_copy(out_v, out_hbm.at[pl.ds(wid * ROWS_PER_W, ROWS_PER_W)])


@functools.partial(jax.jit, donate_argnums=())
def kernel(X, W):
    X2 = X.astype(jnp.int32).reshape(BATCH * 2, HALF)
    mesh = plsc.VectorSubcoreMesh(core_axis_name="c", subcore_axis_name="s")
    k = pl.kernel(
        _body,
        mesh=mesh,
        out_type=jax.ShapeDtypeStruct((BATCH, EMB_DIM), jnp.float32),
        scratch_types=[
            pltpu.VMEM((IDX_CHUNK, HALF), jnp.int32),
            [pltpu.VMEM((HALF, EMB_DIM), jnp.float32) for _ in range(NBUF)],
            pltpu.VMEM((ROWS_PER_W, EMB_DIM), jnp.float32),
            pltpu.VMEM((8, EMB_DIM), jnp.float32),
            pltpu.VMEM((16,), jnp.int32),
            [pltpu.SemaphoreType.DMA for _ in range(NBUF)],
        ],
        compiler_params=pltpu.CompilerParams(
            needs_layout_passes=False, use_tc_tiling_on_sc=False
        ),
    )
    return k(X2, W)
